# Initial kernel scaffold; baseline (speedup 1.0000x reference)
import functools

import jax
import jax.numpy as jnp
from jax import lax
from jax.experimental import pallas as pl
from jax.experimental.pallas import tpu as pltpu
from jax.experimental.pallas import tpu_sc as plsc

NC = 2
NS = 16
NW = NC * NS
L = 16
NFEAT = 18


def _features(x1, y1, z1, x2, y2, z2):
  q1 = x1 * x1 + y1 * y1 + z1 * z1
  q2 = x2 * x2 + y2 * y2 + z2 * z2
  return (
      x1, y1, z1, x2, y2, z2, q1, q2,
      x1 * x2, x1 * y2, x1 * z2,
      y1 * x2, y1 * y2, y1 * z2,
      z1 * x2, z1 * y2, z1 * z2,
  )


def _sc_stage(c1f, c2f, ids, M, T, interpret=False):
  N = ids.shape[0]
  assert N % NW == 0
  rpw = N // NW
  C = L * T
  assert rpw % C == 0
  nch = rpw // C
  assert nch % 2 == 0

  mesh = plsc.VectorSubcoreMesh(
      core_axis_name="c", subcore_axis_name="s",
      num_cores=NC, num_subcores=NS)

  def body(c1_hbm, c2_hbm, ids_hbm, out_hbm,
           c1v, c2v, idsv, accr, scr, sem0, sem1):
    wid = lax.axis_index("s") * NC + lax.axis_index("c")
    base = wid * rpw

    iota = lax.iota(jnp.int32, L)
    iotaT = iota * T
    iota3T = iota * (3 * T)
    zeros = jnp.zeros((L,), jnp.float32)
    sems = (sem0, sem1)

    def zb(i, _):
      accr[pl.ds(i * L, L)] = zeros
      return 0
    lax.fori_loop(0, (NFEAT * M) // L, zb, 0)

    def issue(ci, b):
      rb = base + ci * C
      pltpu.async_copy(c1_hbm.at[pl.ds(rb * 3, C * 3)], c1v.at[b], sems[b])
      pltpu.async_copy(c2_hbm.at[pl.ds(rb * 3, C * 3)], c2v.at[b], sems[b])
      pltpu.async_copy(ids_hbm.at[pl.ds(rb, C)], idsv.at[b], sems[b])

    def wait(b):
      pltpu.make_async_copy(c1_hbm.at[pl.ds(0, C * 3)], c1v.at[b], sems[b]).wait()
      pltpu.make_async_copy(c2_hbm.at[pl.ds(0, C * 3)], c2v.at[b], sems[b]).wait()
      pltpu.make_async_copy(ids_hbm.at[pl.ds(0, C)], idsv.at[b], sems[b]).wait()

    def process(b):
      c1b, c2b, idb = c1v.at[b], c2v.at[b], idsv.at[b]
      g0 = plsc.load_gather(idb, [iotaT])

      def step(t, carry):
        gp, accs = carry
        t3 = t * 3
        g = plsc.load_gather(idb, [iotaT + t])
        x1 = plsc.load_gather(c1b, [iota3T + t3])
        y1 = plsc.load_gather(c1b, [iota3T + (t3 + 1)])
        z1 = plsc.load_gather(c1b, [iota3T + (t3 + 2)])
        x2 = plsc.load_gather(c2b, [iota3T + t3])
        y2 = plsc.load_gather(c2b, [iota3T + (t3 + 1)])
        z2 = plsc.load_gather(c2b, [iota3T + (t3 + 2)])
        m = g != gp
        anyf = jnp.max(m.astype(jnp.int32))

        @pl.when(anyf > 0)
        def _():
          for k in range(NFEAT):
            plsc.addupdate_scatter(accr, [gp + (k * M)], accs[k], mask=m)

        feats = _features(x1, y1, z1, x2, y2, z2)
        new0 = jnp.where(m, 0.0, accs[0]) + 1.0
        new = (new0,) + tuple(
            jnp.where(m, 0.0, a) + f for a, f in zip(accs[1:], feats))
        return (g, new)

      gl, accs = lax.fori_loop(0, T, step, (g0, (zeros,) * NFEAT))

      scr[pl.ds(1, L)] = gl
      gp_l = scr[pl.ds(0, L)]
      gn_l = scr[pl.ds(2, L)]
      m_end = (gl != gn_l) | (iota == (L - 1))
      m_st = (gl != gp_l) & (iota > 0)
      for k in range(NFEAT):
        s = plsc.cumsum(accs[k])
        e = accs[k] - s
        plsc.addupdate_scatter(accr, [gl + (k * M)], s, mask=m_end)
        plsc.addupdate_scatter(accr, [gl + (k * M)], e, mask=m_st)

    issue(0, 0)
    issue(1, 1)

    def pair(k2, _):
      for b in (0, 1):
        ci = k2 * 2 + b
        wait(b)
        process(b)

        @pl.when(ci + 2 < nch)
        def _():
          issue(ci + 2, b)
      return 0

    lax.fori_loop(0, nch // 2, pair, 0)
    pltpu.sync_copy(accr, out_hbm.at[wid])

  f = pl.kernel(
      body,
      out_type=jax.ShapeDtypeStruct((NW, NFEAT * M), jnp.float32),
      mesh=mesh,
      scratch_types=[
          pltpu.VMEM((2, C * 3), jnp.float32),
          pltpu.VMEM((2, C * 3), jnp.float32),
          pltpu.VMEM((2, C), jnp.int32),
          pltpu.VMEM((NFEAT * M,), jnp.float32),
          pltpu.VMEM((L + 2,), jnp.int32),
          pltpu.SemaphoreType.DMA,
          pltpu.SemaphoreType.DMA,
      ],
      interpret=interpret,
  )
  return f(c1f, c2f, ids)


def _tc_body(p_ref, o_ref):
  x = p_ref[...]
  s = x[0]
  for w in range(1, NW):
    s = s + x[w]

  cnt = jnp.maximum(s[0], 1.0)
  inv = 1.0 / cnt
  mu1 = [s[1] * inv, s[2] * inv, s[3] * inv]
  mu2 = [s[4] * inv, s[5] * inv, s[6] * inv]
  n1 = mu1[0] * mu1[0] + mu1[1] * mu1[1] + mu1[2] * mu1[2]
  n2 = mu2[0] * mu2[0] + mu2[1] * mu2[1] + mu2[2] * mu2[2]
  third = jnp.float32(1.0 / 3.0)
  var1 = (s[7] * inv - n1) * third
  var2 = (s[8] * inv - n2) * third

  cov = [[s[9 + 3 * i + j] * inv - mu1[i] * mu2[j] for j in range(3)]
         for i in range(3)]
  det = (cov[0][0] * (cov[1][1] * cov[2][2] - cov[1][2] * cov[2][1])
         - cov[0][1] * (cov[1][0] * cov[2][2] - cov[1][2] * cov[2][0])
         + cov[0][2] * (cov[1][0] * cov[2][1] - cov[1][1] * cov[2][0]))

  b = [[None] * 3 for _ in range(3)]
  for a_ in range(3):
    for c_ in range(a_, 3):
      v = (cov[0][a_] * cov[0][c_] + cov[1][a_] * cov[1][c_]
           + cov[2][a_] * cov[2][c_])
      b[a_][c_] = v
      b[c_][a_] = v

  zero = jnp.zeros_like(cnt)
  for _ in range(8):
    for (p, q) in ((0, 1), (0, 2), (1, 2)):
      r = 3 - p - q
      app, aqq, apq = b[p][p], b[q][q], b[p][q]
      arp, arq = b[r][p], b[r][q]
      theta = (aqq - app) / (2.0 * apq)
      t = jnp.sign(theta) / (jnp.abs(theta) + jnp.sqrt(1.0 + theta * theta))
      t = jnp.where(theta == 0.0, 1.0, t)
      t = jnp.where(apq == 0.0, 0.0, t)
      c = lax.rsqrt(1.0 + t * t)
      sn = t * c
      b[p][p] = app - t * apq
      b[q][q] = aqq + t * apq
      b[p][q] = zero
      b[q][p] = zero
      nrp = c * arp - sn * arq
      nrq = sn * arp + c * arq
      b[r][p] = nrp
      b[p][r] = nrp
      b[r][q] = nrq
      b[q][r] = nrq

  d0, d1, d2 = b[0][0], b[1][1], b[2][2]
  lmin = jnp.minimum(jnp.minimum(d0, d1), d2)
  sq = lambda v: jnp.sqrt(jnp.maximum(v, 0.0))
  ssum = sq(d0) + sq(d1) + sq(d2)
  sig = jnp.where(det < 0.0, ssum - 2.0 * sq(lmin), ssum) * third
  o_ref[...] = var1 + var2 - 2.0 * sig


def _impl(coords1, coords2, segment_ids, M, T, interpret=False):
  part = _sc_stage(coords1.reshape(-1), coords2.reshape(-1), segment_ids,
                   M, T, interpret=interpret)
  p4 = part.reshape(NW, NFEAT, 8, M // 8)
  out = pl.pallas_call(
      _tc_body,
      out_shape=jax.ShapeDtypeStruct((8, M // 8), jnp.float32),
      interpret=interpret,
  )(p4)
  return out.reshape(M)


def kernel(coords1, coords2, segment_ids):
  return _impl(coords1, coords2, segment_ids, 1024, 125)


# speedup vs baseline: 39.0286x; 39.0286x over previous
import functools

import jax
import jax.numpy as jnp
from jax import lax
from jax.experimental import pallas as pl
from jax.experimental.pallas import tpu as pltpu
from jax.experimental.pallas import tpu_sc as plsc

NC = 2
NS = 16
NW = NC * NS
L = 16
NFEAT = 18


def _features(x1, y1, z1, x2, y2, z2):
  q1 = x1 * x1 + y1 * y1 + z1 * z1
  q2 = x2 * x2 + y2 * y2 + z2 * z2
  return (
      x1, y1, z1, x2, y2, z2, q1, q2,
      x1 * x2, x1 * y2, x1 * z2,
      y1 * x2, y1 * y2, y1 * z2,
      z1 * x2, z1 * y2, z1 * z2,
  )


def _sc_stage(c1f, c2f, ids, M, T, interpret=False):
  N = ids.shape[0]
  assert N % NW == 0
  rpw = N // NW
  C = L * T
  assert rpw % C == 0
  nch = rpw // C
  assert nch % 2 == 0

  mesh = plsc.VectorSubcoreMesh(
      core_axis_name="c", subcore_axis_name="s",
      num_cores=NC, num_subcores=NS)

  def body(c1_hbm, c2_hbm, ids_hbm, out_hbm,
           c1v0, c1v1, c2v0, c2v1, idsv0, idsv1, accr, scr, sem0, sem1):
    c1v = (c1v0, c1v1)
    c2v = (c2v0, c2v1)
    idsv = (idsv0, idsv1)
    wid = lax.axis_index("s") * NC + lax.axis_index("c")
    base = wid * rpw

    iota = lax.iota(jnp.int32, L)
    iotaT = iota * T
    iota3T = iota * (3 * T)
    zeros = jnp.zeros((L,), jnp.float32)
    sems = (sem0, sem1)

    def zb(i, _):
      accr[pl.ds(i * L, L)] = zeros
      return 0
    lax.fori_loop(0, (NFEAT * M) // L, zb, 0)

    def issue(ci, b):
      rb = base + ci * C
      pltpu.async_copy(c1_hbm.at[pl.ds(rb * 3, C * 3)], c1v[b], sems[b])
      pltpu.async_copy(c2_hbm.at[pl.ds(rb * 3, C * 3)], c2v[b], sems[b])
      pltpu.async_copy(ids_hbm.at[pl.ds(rb, C)], idsv[b], sems[b])

    def wait(b):
      pltpu.make_async_copy(c1_hbm.at[pl.ds(0, C * 3)], c1v[b], sems[b]).wait()
      pltpu.make_async_copy(c2_hbm.at[pl.ds(0, C * 3)], c2v[b], sems[b]).wait()
      pltpu.make_async_copy(ids_hbm.at[pl.ds(0, C)], idsv[b], sems[b]).wait()

    def process(b):
      c1b, c2b, idb = c1v[b], c2v[b], idsv[b]
      g0 = plsc.load_gather(idb, [iotaT])

      def step(t, carry):
        gp, accs = carry
        t3 = t * 3
        g = plsc.load_gather(idb, [iotaT + t])
        x1 = plsc.load_gather(c1b, [iota3T + t3])
        y1 = plsc.load_gather(c1b, [iota3T + (t3 + 1)])
        z1 = plsc.load_gather(c1b, [iota3T + (t3 + 2)])
        x2 = plsc.load_gather(c2b, [iota3T + t3])
        y2 = plsc.load_gather(c2b, [iota3T + (t3 + 1)])
        z2 = plsc.load_gather(c2b, [iota3T + (t3 + 2)])
        m = g != gp
        anyf = jnp.max(m.astype(jnp.int32))

        @pl.when(anyf > 0)
        def _():
          for k in range(NFEAT):
            plsc.addupdate_scatter(accr, [gp + (k * M)], accs[k], mask=m)

        feats = _features(x1, y1, z1, x2, y2, z2)
        new0 = jnp.where(m, 0.0, accs[0]) + 1.0
        new = (new0,) + tuple(
            jnp.where(m, 0.0, a) + f for a, f in zip(accs[1:], feats))
        return (g, new)

      gl, accs = lax.fori_loop(0, T, step, (g0, (zeros,) * NFEAT))

      scr[pl.ds(1, L)] = gl
      gp_l = scr[pl.ds(0, L)]
      gn_l = scr[pl.ds(2, L)]
      m_end = (gl != gn_l) | (iota == (L - 1))
      m_st = (gl != gp_l) & (iota > 0)
      for k in range(NFEAT):
        s = plsc.cumsum(accs[k])
        e = accs[k] - s
        plsc.addupdate_scatter(accr, [gl + (k * M)], s, mask=m_end)
        plsc.addupdate_scatter(accr, [gl + (k * M)], e, mask=m_st)

    issue(0, 0)
    issue(1, 1)

    def pair(k2, _):
      for b in (0, 1):
        ci = k2 * 2 + b
        wait(b)
        process(b)

        @pl.when(ci + 2 < nch)
        def _():
          issue(ci + 2, b)
      return 0

    lax.fori_loop(0, nch // 2, pair, 0)
    pltpu.sync_copy(accr, out_hbm.at[wid])

  f = pl.kernel(
      body,
      out_type=jax.ShapeDtypeStruct((NW, NFEAT * M), jnp.float32),
      mesh=mesh,
      compiler_params=pltpu.CompilerParams(
          use_tc_tiling_on_sc=False, needs_layout_passes=False),
      scratch_types=[
          pltpu.VMEM((C * 3,), jnp.float32),
          pltpu.VMEM((C * 3,), jnp.float32),
          pltpu.VMEM((C * 3,), jnp.float32),
          pltpu.VMEM((C * 3,), jnp.float32),
          pltpu.VMEM((C,), jnp.int32),
          pltpu.VMEM((C,), jnp.int32),
          pltpu.VMEM((NFEAT * M,), jnp.float32),
          pltpu.VMEM((L + 2,), jnp.int32),
          pltpu.SemaphoreType.DMA,
          pltpu.SemaphoreType.DMA,
      ],
      interpret=interpret,
  )
  return f(c1f, c2f, ids)


def _tc_body(p_ref, o_ref):
  x = p_ref[...]
  s = x[0]
  for w in range(1, NW):
    s = s + x[w]

  cnt = jnp.maximum(s[0], 1.0)
  inv = 1.0 / cnt
  mu1 = [s[1] * inv, s[2] * inv, s[3] * inv]
  mu2 = [s[4] * inv, s[5] * inv, s[6] * inv]
  n1 = mu1[0] * mu1[0] + mu1[1] * mu1[1] + mu1[2] * mu1[2]
  n2 = mu2[0] * mu2[0] + mu2[1] * mu2[1] + mu2[2] * mu2[2]
  third = jnp.float32(1.0 / 3.0)
  var1 = (s[7] * inv - n1) * third
  var2 = (s[8] * inv - n2) * third

  cov = [[s[9 + 3 * i + j] * inv - mu1[i] * mu2[j] for j in range(3)]
         for i in range(3)]
  det = (cov[0][0] * (cov[1][1] * cov[2][2] - cov[1][2] * cov[2][1])
         - cov[0][1] * (cov[1][0] * cov[2][2] - cov[1][2] * cov[2][0])
         + cov[0][2] * (cov[1][0] * cov[2][1] - cov[1][1] * cov[2][0]))

  b = [[None] * 3 for _ in range(3)]
  for a_ in range(3):
    for c_ in range(a_, 3):
      v = (cov[0][a_] * cov[0][c_] + cov[1][a_] * cov[1][c_]
           + cov[2][a_] * cov[2][c_])
      b[a_][c_] = v
      b[c_][a_] = v

  zero = jnp.zeros_like(cnt)
  for _ in range(8):
    for (p, q) in ((0, 1), (0, 2), (1, 2)):
      r = 3 - p - q
      app, aqq, apq = b[p][p], b[q][q], b[p][q]
      arp, arq = b[r][p], b[r][q]
      theta = (aqq - app) / (2.0 * apq)
      t = jnp.sign(theta) / (jnp.abs(theta) + jnp.sqrt(1.0 + theta * theta))
      t = jnp.where(theta == 0.0, 1.0, t)
      t = jnp.where(apq == 0.0, 0.0, t)
      c = lax.rsqrt(1.0 + t * t)
      sn = t * c
      b[p][p] = app - t * apq
      b[q][q] = aqq + t * apq
      b[p][q] = zero
      b[q][p] = zero
      nrp = c * arp - sn * arq
      nrq = sn * arp + c * arq
      b[r][p] = nrp
      b[p][r] = nrp
      b[r][q] = nrq
      b[q][r] = nrq

  d0, d1, d2 = b[0][0], b[1][1], b[2][2]
  lmin = jnp.minimum(jnp.minimum(d0, d1), d2)
  sq = lambda v: jnp.sqrt(jnp.maximum(v, 0.0))
  ssum = sq(d0) + sq(d1) + sq(d2)
  sig = jnp.where(det < 0.0, ssum - 2.0 * sq(lmin), ssum) * third
  o_ref[...] = var1 + var2 - 2.0 * sig


def _impl(coords1, coords2, segment_ids, M, T, interpret=False):
  part = _sc_stage(coords1.reshape(-1), coords2.reshape(-1), segment_ids,
                   M, T, interpret=interpret)
  p4 = part.reshape(NW, NFEAT, 8, M // 8)
  out = pl.pallas_call(
      _tc_body,
      out_shape=jax.ShapeDtypeStruct((8, M // 8), jnp.float32),
      interpret=interpret,
  )(p4)
  return out.reshape(M)


def kernel(coords1, coords2, segment_ids):
  return _impl(coords1, coords2, segment_ids, 1024, 125)


# speedup vs baseline: 875.7873x; 22.4396x over previous
import functools

import jax
import jax.numpy as jnp
from jax import lax
from jax.experimental import pallas as pl
from jax.experimental.pallas import tpu as pltpu
from jax.experimental.pallas import tpu_sc as plsc

NC = 2
NS = 16
NW = NC * NS
L = 16
NFEAT = 18


def _features(x1, y1, z1, x2, y2, z2):
  q1 = x1 * x1 + y1 * y1 + z1 * z1
  q2 = x2 * x2 + y2 * y2 + z2 * z2
  return (
      x1, y1, z1, x2, y2, z2, q1, q2,
      x1 * x2, x1 * y2, x1 * z2,
      y1 * x2, y1 * y2, y1 * z2,
      z1 * x2, z1 * y2, z1 * z2,
  )


def _sc_stage(x1a, y1a, z1a, x2a, y2a, z2a, ids, M, T, interpret=False):
  N = ids.shape[0]
  assert N % NW == 0
  rpw = N // NW
  C = L * T
  assert rpw % C == 0
  nch = rpw // C
  assert nch % 2 == 0

  mesh = plsc.VectorSubcoreMesh(
      core_axis_name="c", subcore_axis_name="s",
      num_cores=NC, num_subcores=NS)

  def body(x1_hbm, y1_hbm, z1_hbm, x2_hbm, y2_hbm, z2_hbm, ids_hbm, out_hbm,
           *scratch):
    cv = tuple(zip(scratch[0:6], scratch[6:12]))
    idsv = (scratch[12], scratch[13])
    accr, scr, sem0, sem1 = scratch[14:]
    hbm = (x1_hbm, y1_hbm, z1_hbm, x2_hbm, y2_hbm, z2_hbm)
    wid = lax.axis_index("s") * NC + lax.axis_index("c")
    base = wid * rpw

    iota = lax.iota(jnp.int32, L)
    iotaT = iota * T
    zeros = jnp.zeros((L,), jnp.float32)
    sems = (sem0, sem1)

    def zb(i, _):
      accr[pl.ds(i * L, L)] = zeros
      return 0
    lax.fori_loop(0, (NFEAT * M) // L, zb, 0)

    def issue(ci, b):
      rb = base + ci * C
      for j in range(6):
        pltpu.async_copy(hbm[j].at[pl.ds(rb, C)], cv[j][b], sems[b])
      pltpu.async_copy(ids_hbm.at[pl.ds(rb, C)], idsv[b], sems[b])

    def wait(b):
      for j in range(6):
        pltpu.make_async_copy(hbm[j].at[pl.ds(0, C)], cv[j][b], sems[b]).wait()
      pltpu.make_async_copy(ids_hbm.at[pl.ds(0, C)], idsv[b], sems[b]).wait()

    def process(b):
      idb = idsv[b]
      g0 = plsc.load_gather(idb, [iotaT])

      def step(t, carry):
        gp, accs = carry
        idx = iotaT + t
        g = plsc.load_gather(idb, [idx])
        x1 = plsc.load_gather(cv[0][b], [idx])
        y1 = plsc.load_gather(cv[1][b], [idx])
        z1 = plsc.load_gather(cv[2][b], [idx])
        x2 = plsc.load_gather(cv[3][b], [idx])
        y2 = plsc.load_gather(cv[4][b], [idx])
        z2 = plsc.load_gather(cv[5][b], [idx])
        m = g != gp
        anyf = jnp.max(m.astype(jnp.int32))

        @pl.when(anyf > 0)
        def _():
          for k in range(NFEAT):
            plsc.addupdate_scatter(accr, [gp + (k * M)], accs[k], mask=m)

        feats = _features(x1, y1, z1, x2, y2, z2)
        new0 = jnp.where(m, 0.0, accs[0]) + 1.0
        new = (new0,) + tuple(
            jnp.where(m, 0.0, a) + f for a, f in zip(accs[1:], feats))
        return (g, new)

      gl, accs = lax.fori_loop(0, T, step, (g0, (zeros,) * NFEAT))

      scr[pl.ds(1, L)] = gl
      gp_l = scr[pl.ds(0, L)]
      gn_l = scr[pl.ds(2, L)]
      m_end = (gl != gn_l) | (iota == (L - 1))
      m_st = (gl != gp_l) & (iota > 0)
      for k in range(NFEAT):
        s = plsc.cumsum(accs[k])
        e = accs[k] - s
        plsc.addupdate_scatter(accr, [gl + (k * M)], s, mask=m_end)
        plsc.addupdate_scatter(accr, [gl + (k * M)], e, mask=m_st)

    issue(0, 0)
    issue(1, 1)

    def pair(k2, _):
      for b in (0, 1):
        ci = k2 * 2 + b
        wait(b)
        process(b)

        @pl.when(ci + 2 < nch)
        def _():
          issue(ci + 2, b)
      return 0

    lax.fori_loop(0, nch // 2, pair, 0)
    pltpu.sync_copy(accr, out_hbm.at[wid])

  f = pl.kernel(
      body,
      out_type=jax.ShapeDtypeStruct((NW, NFEAT * M), jnp.float32),
      mesh=mesh,
      compiler_params=pltpu.CompilerParams(
          use_tc_tiling_on_sc=False, needs_layout_passes=False),
      scratch_types=(
          [pltpu.VMEM((C,), jnp.float32) for _ in range(12)]
          + [
              pltpu.VMEM((C,), jnp.int32),
              pltpu.VMEM((C,), jnp.int32),
              pltpu.VMEM((NFEAT * M,), jnp.float32),
              pltpu.VMEM((L + 2,), jnp.int32),
              pltpu.SemaphoreType.DMA,
              pltpu.SemaphoreType.DMA,
          ]
      ),
      interpret=interpret,
  )
  return f(x1a, y1a, z1a, x2a, y2a, z2a, ids)


def _tc_body(p_ref, o_ref):
  x = p_ref[...]
  s = x[0]
  for w in range(1, NW):
    s = s + x[w]

  cnt = jnp.maximum(s[0], 1.0)
  inv = 1.0 / cnt
  mu1 = [s[1] * inv, s[2] * inv, s[3] * inv]
  mu2 = [s[4] * inv, s[5] * inv, s[6] * inv]
  n1 = mu1[0] * mu1[0] + mu1[1] * mu1[1] + mu1[2] * mu1[2]
  n2 = mu2[0] * mu2[0] + mu2[1] * mu2[1] + mu2[2] * mu2[2]
  third = jnp.float32(1.0 / 3.0)
  var1 = (s[7] * inv - n1) * third
  var2 = (s[8] * inv - n2) * third

  cov = [[s[9 + 3 * i + j] * inv - mu1[i] * mu2[j] for j in range(3)]
         for i in range(3)]
  det = (cov[0][0] * (cov[1][1] * cov[2][2] - cov[1][2] * cov[2][1])
         - cov[0][1] * (cov[1][0] * cov[2][2] - cov[1][2] * cov[2][0])
         + cov[0][2] * (cov[1][0] * cov[2][1] - cov[1][1] * cov[2][0]))

  b = [[None] * 3 for _ in range(3)]
  for a_ in range(3):
    for c_ in range(a_, 3):
      v = (cov[0][a_] * cov[0][c_] + cov[1][a_] * cov[1][c_]
           + cov[2][a_] * cov[2][c_])
      b[a_][c_] = v
      b[c_][a_] = v

  zero = jnp.zeros_like(cnt)
  for _ in range(8):
    for (p, q) in ((0, 1), (0, 2), (1, 2)):
      r = 3 - p - q
      app, aqq, apq = b[p][p], b[q][q], b[p][q]
      arp, arq = b[r][p], b[r][q]
      theta = (aqq - app) / (2.0 * apq)
      t = jnp.sign(theta) / (jnp.abs(theta) + jnp.sqrt(1.0 + theta * theta))
      t = jnp.where(theta == 0.0, 1.0, t)
      t = jnp.where(apq == 0.0, 0.0, t)
      c = lax.rsqrt(1.0 + t * t)
      sn = t * c
      b[p][p] = app - t * apq
      b[q][q] = aqq + t * apq
      b[p][q] = zero
      b[q][p] = zero
      nrp = c * arp - sn * arq
      nrq = sn * arp + c * arq
      b[r][p] = nrp
      b[p][r] = nrp
      b[r][q] = nrq
      b[q][r] = nrq

  d0, d1, d2 = b[0][0], b[1][1], b[2][2]
  lmin = jnp.minimum(jnp.minimum(d0, d1), d2)
  sq = lambda v: jnp.sqrt(jnp.maximum(v, 0.0))
  ssum = sq(d0) + sq(d1) + sq(d2)
  sig = jnp.where(det < 0.0, ssum - 2.0 * sq(lmin), ssum) * third
  o_ref[...] = var1 + var2 - 2.0 * sig


def _impl(coords1, coords2, segment_ids, M, T, interpret=False):
  part = _sc_stage(coords1[:, 0], coords1[:, 1], coords1[:, 2],
                   coords2[:, 0], coords2[:, 1], coords2[:, 2],
                   segment_ids, M, T, interpret=interpret)
  p4 = part.reshape(NW, NFEAT, 8, M // 8)
  out = pl.pallas_call(
      _tc_body,
      out_shape=jax.ShapeDtypeStruct((8, M // 8), jnp.float32),
      interpret=interpret,
  )(p4)
  return out.reshape(M)


def kernel(coords1, coords2, segment_ids):
  return _impl(coords1, coords2, segment_ids, 1024, 125)


# speedup vs baseline: 1230.9456x; 1.4055x over previous
import functools

import jax
import jax.numpy as jnp
from jax import lax
from jax.experimental import pallas as pl
from jax.experimental.pallas import tpu as pltpu
from jax.experimental.pallas import tpu_sc as plsc

NC = 2
NS = 16
NW = NC * NS
L = 16
NFEAT = 18


def _features(x1, y1, z1, x2, y2, z2):
  q1 = x1 * x1 + y1 * y1 + z1 * z1
  q2 = x2 * x2 + y2 * y2 + z2 * z2
  return (
      x1, y1, z1, x2, y2, z2, q1, q2,
      x1 * x2, x1 * y2, x1 * z2,
      y1 * x2, y1 * y2, y1 * z2,
      z1 * x2, z1 * y2, z1 * z2,
  )


def _sc_stage(x1a, y1a, z1a, x2a, y2a, z2a, ids, M, T, interpret=False):
  N = ids.shape[0]
  assert N % NW == 0
  rpw = N // NW
  C = L * T
  assert rpw % C == 0
  nch = rpw // C
  assert nch % 2 == 0

  mesh = plsc.VectorSubcoreMesh(
      core_axis_name="c", subcore_axis_name="s",
      num_cores=NC, num_subcores=NS)

  def body(x1_hbm, y1_hbm, z1_hbm, x2_hbm, y2_hbm, z2_hbm, ids_hbm, out_hbm,
           *scratch):
    cv = tuple(zip(scratch[0:6], scratch[6:12]))
    idsv = (scratch[12], scratch[13])
    accr, scr, sem0, sem1 = scratch[14:]
    hbm = (x1_hbm, y1_hbm, z1_hbm, x2_hbm, y2_hbm, z2_hbm)
    wid = lax.axis_index("s") * NC + lax.axis_index("c")
    base = wid * rpw

    iota = lax.iota(jnp.int32, L)
    iotaT = iota * T
    zeros = jnp.zeros((L,), jnp.float32)
    sems = (sem0, sem1)

    def zb(i, _):
      accr[pl.ds(i * L, L)] = zeros
      return 0
    lax.fori_loop(0, (NFEAT * M) // L, zb, 0)

    def issue(ci, b):
      rb = base + ci * C
      for j in range(6):
        pltpu.async_copy(hbm[j].at[pl.ds(rb, C)], cv[j][b], sems[b])
      pltpu.async_copy(ids_hbm.at[pl.ds(rb, C)], idsv[b], sems[b])

    def wait(b):
      for j in range(6):
        pltpu.make_async_copy(hbm[j].at[pl.ds(0, C)], cv[j][b], sems[b]).wait()
      pltpu.make_async_copy(ids_hbm.at[pl.ds(0, C)], idsv[b], sems[b]).wait()

    def process(b):
      idb = idsv[b]
      g0 = plsc.load_gather(idb, [iotaT])

      def step(t, carry):
        gp, accs = carry
        idx = iotaT + t
        g = plsc.load_gather(idb, [idx])
        x1 = plsc.load_gather(cv[0][b], [idx])
        y1 = plsc.load_gather(cv[1][b], [idx])
        z1 = plsc.load_gather(cv[2][b], [idx])
        x2 = plsc.load_gather(cv[3][b], [idx])
        y2 = plsc.load_gather(cv[4][b], [idx])
        z2 = plsc.load_gather(cv[5][b], [idx])
        m = g != gp
        for k in range(NFEAT):
          plsc.addupdate_scatter(accr.at[pl.ds(k * M, M)], [gp], accs[k],
                                 mask=m)
        keep = jnp.where(m, 0.0, 1.0)
        feats = _features(x1, y1, z1, x2, y2, z2)
        new = (accs[0] * keep + 1.0,) + tuple(
            a * keep + f for a, f in zip(accs[1:], feats))
        return (g, new)

      gl, accs = lax.fori_loop(0, T, step, (g0, (zeros,) * NFEAT),
                               unroll=2)

      scr[pl.ds(1, L)] = gl
      gp_l = scr[pl.ds(0, L)]
      gn_l = scr[pl.ds(2, L)]
      m_end = (gl != gn_l) | (iota == (L - 1))
      m_st = (gl != gp_l) & (iota > 0)
      for k in range(NFEAT):
        s = plsc.cumsum(accs[k])
        e = accs[k] - s
        plsc.addupdate_scatter(accr.at[pl.ds(k * M, M)], [gl], s, mask=m_end)
        plsc.addupdate_scatter(accr.at[pl.ds(k * M, M)], [gl], e, mask=m_st)

    issue(0, 0)
    issue(1, 1)

    def pair(k2, _):
      for b in (0, 1):
        ci = k2 * 2 + b
        wait(b)
        process(b)

        @pl.when(ci + 2 < nch)
        def _():
          issue(ci + 2, b)
      return 0

    lax.fori_loop(0, nch // 2, pair, 0)
    pltpu.sync_copy(accr, out_hbm.at[wid])

  f = pl.kernel(
      body,
      out_type=jax.ShapeDtypeStruct((NW, NFEAT * M), jnp.float32),
      mesh=mesh,
      compiler_params=pltpu.CompilerParams(
          use_tc_tiling_on_sc=False, needs_layout_passes=False),
      scratch_types=(
          [pltpu.VMEM((C,), jnp.float32) for _ in range(12)]
          + [
              pltpu.VMEM((C,), jnp.int32),
              pltpu.VMEM((C,), jnp.int32),
              pltpu.VMEM((NFEAT * M,), jnp.float32),
              pltpu.VMEM((L + 2,), jnp.int32),
              pltpu.SemaphoreType.DMA,
              pltpu.SemaphoreType.DMA,
          ]
      ),
      interpret=interpret,
  )
  return f(x1a, y1a, z1a, x2a, y2a, z2a, ids)


def _tc_body(p_ref, o_ref):
  x = p_ref[...]
  s = x[0]
  for w in range(1, NW):
    s = s + x[w]

  cnt = jnp.maximum(s[0], 1.0)
  inv = 1.0 / cnt
  mu1 = [s[1] * inv, s[2] * inv, s[3] * inv]
  mu2 = [s[4] * inv, s[5] * inv, s[6] * inv]
  n1 = mu1[0] * mu1[0] + mu1[1] * mu1[1] + mu1[2] * mu1[2]
  n2 = mu2[0] * mu2[0] + mu2[1] * mu2[1] + mu2[2] * mu2[2]
  third = jnp.float32(1.0 / 3.0)
  var1 = (s[7] * inv - n1) * third
  var2 = (s[8] * inv - n2) * third

  cov = [[s[9 + 3 * i + j] * inv - mu1[i] * mu2[j] for j in range(3)]
         for i in range(3)]
  det = (cov[0][0] * (cov[1][1] * cov[2][2] - cov[1][2] * cov[2][1])
         - cov[0][1] * (cov[1][0] * cov[2][2] - cov[1][2] * cov[2][0])
         + cov[0][2] * (cov[1][0] * cov[2][1] - cov[1][1] * cov[2][0]))

  b = [[None] * 3 for _ in range(3)]
  for a_ in range(3):
    for c_ in range(a_, 3):
      v = (cov[0][a_] * cov[0][c_] + cov[1][a_] * cov[1][c_]
           + cov[2][a_] * cov[2][c_])
      b[a_][c_] = v
      b[c_][a_] = v

  zero = jnp.zeros_like(cnt)
  for _ in range(8):
    for (p, q) in ((0, 1), (0, 2), (1, 2)):
      r = 3 - p - q
      app, aqq, apq = b[p][p], b[q][q], b[p][q]
      arp, arq = b[r][p], b[r][q]
      theta = (aqq - app) / (2.0 * apq)
      t = jnp.sign(theta) / (jnp.abs(theta) + jnp.sqrt(1.0 + theta * theta))
      t = jnp.where(theta == 0.0, 1.0, t)
      t = jnp.where(apq == 0.0, 0.0, t)
      c = lax.rsqrt(1.0 + t * t)
      sn = t * c
      b[p][p] = app - t * apq
      b[q][q] = aqq + t * apq
      b[p][q] = zero
      b[q][p] = zero
      nrp = c * arp - sn * arq
      nrq = sn * arp + c * arq
      b[r][p] = nrp
      b[p][r] = nrp
      b[r][q] = nrq
      b[q][r] = nrq

  d0, d1, d2 = b[0][0], b[1][1], b[2][2]
  lmin = jnp.minimum(jnp.minimum(d0, d1), d2)
  sq = lambda v: jnp.sqrt(jnp.maximum(v, 0.0))
  ssum = sq(d0) + sq(d1) + sq(d2)
  sig = jnp.where(det < 0.0, ssum - 2.0 * sq(lmin), ssum) * third
  o_ref[...] = var1 + var2 - 2.0 * sig


def _impl(coords1, coords2, segment_ids, M, T, interpret=False):
  part = _sc_stage(coords1[:, 0], coords1[:, 1], coords1[:, 2],
                   coords2[:, 0], coords2[:, 1], coords2[:, 2],
                   segment_ids, M, T, interpret=interpret)
  p4 = part.reshape(NW, NFEAT, 8, M // 8)
  out = pl.pallas_call(
      _tc_body,
      out_shape=jax.ShapeDtypeStruct((8, M // 8), jnp.float32),
      interpret=interpret,
  )(p4)
  return out.reshape(M)


def kernel(coords1, coords2, segment_ids):
  return _impl(coords1, coords2, segment_ids, 1024, 125)


# speedup vs baseline: 1632.5533x; 1.3263x over previous
import functools

import jax
import jax.numpy as jnp
from jax import lax
from jax.experimental import pallas as pl
from jax.experimental.pallas import tpu as pltpu
from jax.experimental.pallas import tpu_sc as plsc

NC = 2
NS = 16
NW = NC * NS
L = 16
NFEAT = 18


def _features(x1, y1, z1, x2, y2, z2):
  q1 = x1 * x1 + y1 * y1 + z1 * z1
  q2 = x2 * x2 + y2 * y2 + z2 * z2
  return (
      x1, y1, z1, x2, y2, z2, q1, q2,
      x1 * x2, x1 * y2, x1 * z2,
      y1 * x2, y1 * y2, y1 * z2,
      z1 * x2, z1 * y2, z1 * z2,
  )


def _sc_stage(x1a, y1a, z1a, x2a, y2a, z2a, ids, M, T, interpret=False):
  N = ids.shape[0]
  assert N % NW == 0
  rpw = N // NW
  C = L * T
  assert rpw % C == 0
  nch = rpw // C
  assert nch % 2 == 0

  mesh = plsc.VectorSubcoreMesh(
      core_axis_name="c", subcore_axis_name="s",
      num_cores=NC, num_subcores=NS)

  def body(x1_hbm, y1_hbm, z1_hbm, x2_hbm, y2_hbm, z2_hbm, ids_hbm, out_hbm,
           *scratch):
    cv = tuple(zip(scratch[0:6], scratch[6:12]))
    idsv = (scratch[12], scratch[13])
    accr, scr, sem0, sem1 = scratch[14:]
    hbm = (x1_hbm, y1_hbm, z1_hbm, x2_hbm, y2_hbm, z2_hbm)
    wid = lax.axis_index("s") * NC + lax.axis_index("c")
    base = wid * rpw

    iota = lax.iota(jnp.int32, L)
    iotaT = iota * T
    zeros = jnp.zeros((L,), jnp.float32)
    sems = (sem0, sem1)

    def zb(i, _):
      accr[pl.ds(i * L, L)] = zeros
      return 0
    lax.fori_loop(0, (NFEAT * M) // L, zb, 0)

    def issue(ci, b):
      rb = base + ci * C
      for j in range(6):
        pltpu.async_copy(hbm[j].at[pl.ds(rb, C)], cv[j][b], sems[b])
      pltpu.async_copy(ids_hbm.at[pl.ds(rb, C)], idsv[b], sems[b])

    def wait(b):
      for j in range(6):
        pltpu.make_async_copy(hbm[j].at[pl.ds(0, C)], cv[j][b], sems[b]).wait()
      pltpu.make_async_copy(ids_hbm.at[pl.ds(0, C)], idsv[b], sems[b]).wait()

    def process(b):
      idb = idsv[b]
      g0 = plsc.load_gather(idb, [iotaT])

      def step(t, carry):
        gp, accs = carry
        idx = iotaT + t
        g = plsc.load_gather(idb, [idx])
        x1 = plsc.load_gather(cv[0][b], [idx])
        y1 = plsc.load_gather(cv[1][b], [idx])
        z1 = plsc.load_gather(cv[2][b], [idx])
        x2 = plsc.load_gather(cv[3][b], [idx])
        y2 = plsc.load_gather(cv[4][b], [idx])
        z2 = plsc.load_gather(cv[5][b], [idx])
        m = g != gp
        for k in range(NFEAT):
          plsc.addupdate_scatter(accr.at[pl.ds(k * M, M)], [gp], accs[k],
                                 mask=m)
        keep = jnp.where(m, 0.0, 1.0)
        feats = _features(x1, y1, z1, x2, y2, z2)
        new = (accs[0] * keep + 1.0,) + tuple(
            a * keep + f for a, f in zip(accs[1:], feats))
        return (g, new)

      gl, accs = lax.fori_loop(0, T, step, (g0, (zeros,) * NFEAT),
                               unroll=2)

      scr[pl.ds(1, L)] = gl
      gp_l = scr[pl.ds(0, L)]
      gn_l = scr[pl.ds(2, L)]
      m_end = (gl != gn_l) | (iota == (L - 1))
      m_st = (gl != gp_l) & (iota > 0)
      for k in range(NFEAT):
        s = plsc.cumsum(accs[k])
        e = accs[k] - s
        plsc.addupdate_scatter(accr.at[pl.ds(k * M, M)], [gl], s, mask=m_end)
        plsc.addupdate_scatter(accr.at[pl.ds(k * M, M)], [gl], e, mask=m_st)

    issue(0, 0)
    issue(1, 1)

    def pair(k2, _):
      for b in (0, 1):
        ci = k2 * 2 + b
        wait(b)
        process(b)

        @pl.when(ci + 2 < nch)
        def _():
          issue(ci + 2, b)
      return 0

    lax.fori_loop(0, nch // 2, pair, 0)
    pltpu.sync_copy(accr, out_hbm.at[wid])

  f = pl.kernel(
      body,
      out_type=jax.ShapeDtypeStruct((NW, NFEAT * M), jnp.float32),
      mesh=mesh,
      compiler_params=pltpu.CompilerParams(
          use_tc_tiling_on_sc=False, needs_layout_passes=False),
      scratch_types=(
          [pltpu.VMEM((C,), jnp.float32) for _ in range(12)]
          + [
              pltpu.VMEM((C,), jnp.int32),
              pltpu.VMEM((C,), jnp.int32),
              pltpu.VMEM((NFEAT * M,), jnp.float32),
              pltpu.VMEM((L + 2,), jnp.int32),
              pltpu.SemaphoreType.DMA,
              pltpu.SemaphoreType.DMA,
          ]
      ),
      interpret=interpret,
  )
  return f(x1a, y1a, z1a, x2a, y2a, z2a, ids)


def _split_body(z1_ref, z2_ref, x1_ref, y1_ref, z1o_ref, x2_ref, y2_ref,
                z2o_ref):
  a = z1_ref[...]
  b = z2_ref[...]
  x1_ref[...] = a[:, 0, :]
  y1_ref[...] = a[:, 1, :]
  z1o_ref[...] = a[:, 2, :]
  x2_ref[...] = b[:, 0, :]
  y2_ref[...] = b[:, 1, :]
  z2o_ref[...] = b[:, 2, :]


def _split(coords1, coords2):
  N = coords1.shape[0]
  nb = N // 128
  z1 = coords1.reshape(nb, 128, 3).transpose(0, 2, 1)
  z2 = coords2.reshape(nb, 128, 3).transpose(0, 2, 1)
  BB = 1000 if nb % 1000 == 0 else nb
  grid = nb // BB
  outs = pl.pallas_call(
      _split_body,
      grid=(grid,),
      in_specs=[pl.BlockSpec((BB, 3, 128), lambda i: (i, 0, 0))] * 2,
      out_specs=[pl.BlockSpec((BB, 128), lambda i: (i, 0))] * 6,
      out_shape=[jax.ShapeDtypeStruct((nb, 128), jnp.float32)] * 6,
  )(z1, z2)
  return tuple(o.reshape(N) for o in outs)


def _tc_body(p_ref, o_ref):
  x = p_ref[...]
  s = x[0]
  for w in range(1, NW):
    s = s + x[w]

  cnt = jnp.maximum(s[0], 1.0)
  inv = 1.0 / cnt
  mu1 = [s[1] * inv, s[2] * inv, s[3] * inv]
  mu2 = [s[4] * inv, s[5] * inv, s[6] * inv]
  n1 = mu1[0] * mu1[0] + mu1[1] * mu1[1] + mu1[2] * mu1[2]
  n2 = mu2[0] * mu2[0] + mu2[1] * mu2[1] + mu2[2] * mu2[2]
  third = jnp.float32(1.0 / 3.0)
  var1 = (s[7] * inv - n1) * third
  var2 = (s[8] * inv - n2) * third

  cov = [[s[9 + 3 * i + j] * inv - mu1[i] * mu2[j] for j in range(3)]
         for i in range(3)]
  det = (cov[0][0] * (cov[1][1] * cov[2][2] - cov[1][2] * cov[2][1])
         - cov[0][1] * (cov[1][0] * cov[2][2] - cov[1][2] * cov[2][0])
         + cov[0][2] * (cov[1][0] * cov[2][1] - cov[1][1] * cov[2][0]))

  b = [[None] * 3 for _ in range(3)]
  for a_ in range(3):
    for c_ in range(a_, 3):
      v = (cov[0][a_] * cov[0][c_] + cov[1][a_] * cov[1][c_]
           + cov[2][a_] * cov[2][c_])
      b[a_][c_] = v
      b[c_][a_] = v

  zero = jnp.zeros_like(cnt)
  for _ in range(8):
    for (p, q) in ((0, 1), (0, 2), (1, 2)):
      r = 3 - p - q
      app, aqq, apq = b[p][p], b[q][q], b[p][q]
      arp, arq = b[r][p], b[r][q]
      theta = (aqq - app) / (2.0 * apq)
      t = jnp.sign(theta) / (jnp.abs(theta) + jnp.sqrt(1.0 + theta * theta))
      t = jnp.where(theta == 0.0, 1.0, t)
      t = jnp.where(apq == 0.0, 0.0, t)
      c = lax.rsqrt(1.0 + t * t)
      sn = t * c
      b[p][p] = app - t * apq
      b[q][q] = aqq + t * apq
      b[p][q] = zero
      b[q][p] = zero
      nrp = c * arp - sn * arq
      nrq = sn * arp + c * arq
      b[r][p] = nrp
      b[p][r] = nrp
      b[r][q] = nrq
      b[q][r] = nrq

  d0, d1, d2 = b[0][0], b[1][1], b[2][2]
  lmin = jnp.minimum(jnp.minimum(d0, d1), d2)
  sq = lambda v: jnp.sqrt(jnp.maximum(v, 0.0))
  ssum = sq(d0) + sq(d1) + sq(d2)
  sig = jnp.where(det < 0.0, ssum - 2.0 * sq(lmin), ssum) * third
  o_ref[...] = var1 + var2 - 2.0 * sig


def _impl(coords1, coords2, segment_ids, M, T, interpret=False):
  comps = _split(coords1, coords2)
  part = _sc_stage(*comps, segment_ids, M, T, interpret=interpret)
  p4 = part.reshape(NW, NFEAT, 8, M // 8)
  out = pl.pallas_call(
      _tc_body,
      out_shape=jax.ShapeDtypeStruct((8, M // 8), jnp.float32),
      interpret=interpret,
  )(p4)
  return out.reshape(M)


def kernel(coords1, coords2, segment_ids):
  return _impl(coords1, coords2, segment_ids, 1024, 125)
